# baseline (speedup 1.0000x reference)
import functools

import jax
import jax.numpy as jnp
from jax import lax
from jax.experimental import pallas as pl
from jax.experimental.pallas import tpu as pltpu
from jax.experimental.pallas import tpu_sc as plsc

N = 10000
D_IN = 128
H = 128
C = 40

NC = 2
NS = 16
NW = NC * NS
L = 16
B = 128
P = 2

RPT = 624
ZR = 24

E_RAW = 320000
E_TOT = E_RAW + N
NCH = P * (-(-E_TOT // (NW * B * P)))
EPW = NCH * B
E_PAD = NW * EPW


_GDN = lax.GatherDimensionNumbers(
    offset_dims=(), collapsed_slice_dims=(0,), start_index_map=(0,))


def _bcast_lane(vec, lane):
    idx = jnp.full((L,), lane, jnp.int32)
    return lax.gather(vec, idx[:, None], _GDN, (1,),
                      mode=lax.GatherScatterMode.PROMISE_IN_BOUNDS)


def _zero_acc(zero_v, acc, s, d):
    z16 = jnp.zeros((L,), jnp.float32)

    def zrow(r, _):
        for k in range(d // L):
            zero_v[r, pl.ds(k * L, L)] = z16
        return 0

    lax.fori_loop(0, ZR, zrow, 0)
    base = s * RPT
    for i in range(RPT // ZR):
        pltpu.sync_copy(zero_v, acc.at[pl.ds(base + i * ZR, ZR)])

    @pl.when(s == NS - 1)
    def _():
        pltpu.sync_copy(zero_v.at[pl.ds(0, N - NS * RPT)],
                        acc.at[pl.ds(NS * RPT, N - NS * RPT)])


def _copy_out(acc, out_hbm, c, s):
    base = s * RPT
    pltpu.sync_copy(acc.at[pl.ds(base, RPT)],
                    out_hbm.at[pl.ds(c * N + base, RPT)])

    @pl.when(s == NS - 1)
    def _():
        tail = N - NS * RPT
        pltpu.sync_copy(acc.at[pl.ds(NS * RPT, tail)],
                        out_hbm.at[pl.ds(c * N + NS * RPT, tail)])


def _make_sc_agg(d):
    mesh = plsc.VectorSubcoreMesh(core_axis_name="c", subcore_axis_name="s")

    def _scale(rows_v, ew_row):
        def grp(m, _):
            vec = ew_row[pl.ds(m * L, L)]
            for l in range(L):
                sc = _bcast_lane(vec, l)
                row = rows_v.at[m * L + l]
                for k in range(d // L):
                    sl = pl.ds(k * L, L)
                    row[sl] = row[sl] * sc
            return 0

        lax.fori_loop(0, B // L, grp, 0)

    @functools.partial(
        pl.kernel,
        out_type=jax.ShapeDtypeStruct((NC * N, d), jnp.float32),
        mesh=mesh,
        scratch_types=[
            pltpu.VMEM((P, B), jnp.int32),
            pltpu.VMEM((P, B), jnp.int32),
            pltpu.VMEM((P, B), jnp.float32),
            pltpu.VMEM((P, B), jnp.int32),
            pltpu.VMEM((P, B), jnp.int32),
            pltpu.VMEM((P, B), jnp.float32),
            pltpu.VMEM((B, d), jnp.float32),
            pltpu.VMEM((B, d), jnp.float32),
            pltpu.VMEM((ZR, d), jnp.float32),
            pltpu.VMEM_SHARED((N, d), jnp.float32),
            pltpu.SemaphoreType.DMA,
            pltpu.SemaphoreType.DMA,
            pltpu.SemaphoreType.DMA,
        ],
    )
    def agg(row_hbm, col_hbm, ew_hbm, x2_hbm, out_hbm,
            rowA, colA, ewA, rowB, colB, ewB, rows0, rows1,
            zero_v, acc, semi, sem0, sem1):
        c = lax.axis_index("c")
        s = lax.axis_index("s")
        wid = c * NS + s

        _zero_acc(zero_v, acc, s, d)
        plsc.subcore_barrier()

        rows = (rows0, rows1)
        sems = (sem0, sem1)
        idxs = ((rowA, colA, ewA), (rowB, colB, ewB))

        def idx_start(blk, slot):
            for src_hbm, dst in zip((row_hbm, col_hbm, ew_hbm), slot):
                pltpu.async_copy(src_hbm.at[wid].at[pl.ds(blk * P, P)], dst,
                                 semi)

        def idx_wait(blk, slot):
            for src_hbm, dst in zip((row_hbm, col_hbm, ew_hbm), slot):
                pltpu.make_async_copy(src_hbm.at[wid].at[pl.ds(blk * P, P)],
                                      dst, semi).wait()

        idx_start(0, idxs[0])
        idx_wait(0, idxs[0])
        for p in range(P):
            pltpu.async_copy(x2_hbm.at[idxs[0][0].at[p]], rows[p], sems[p])
        idx_start(1, idxs[1])

        nblk = NCH // P

        def block(i, _):
            for q in range(2):
                @pl.when(i % 2 == q)
                def _():
                    cur = idxs[q]
                    nxt = idxs[1 - q]
                    for p in range(P):
                        pltpu.make_async_copy(
                            x2_hbm.at[cur[0].at[p]], rows[p], sems[p]).wait()
                        _scale(rows[p], cur[2].at[p])
                        pltpu.sync_copy(rows[p], acc.at[cur[1].at[p]],
                                        add=True)

                        @pl.when(i + 1 < nblk)
                        def _():
                            if p == 0:
                                idx_wait(i + 1, nxt)
                            pltpu.async_copy(
                                x2_hbm.at[nxt[0].at[p]], rows[p], sems[p])

                    @pl.when(i + 2 < nblk)
                    def _():
                        idx_start(i + 2, cur)
            return 0

        lax.fori_loop(0, nblk, block, 0)

        plsc.subcore_barrier()
        _copy_out(acc, out_hbm, c, s)

    return agg


def _make_sc_deg():
    d = H
    mesh = plsc.VectorSubcoreMesh(core_axis_name="c", subcore_axis_name="s")

    @functools.partial(
        pl.kernel,
        out_type=jax.ShapeDtypeStruct((NC * N, d), jnp.float32),
        mesh=mesh,
        scratch_types=[
            pltpu.VMEM((NCH, B), jnp.int32),
            pltpu.VMEM((NCH, B), jnp.float32),
            pltpu.VMEM((B, d), jnp.float32),
            pltpu.VMEM((ZR, d), jnp.float32),
            pltpu.VMEM_SHARED((N, d), jnp.float32),
            pltpu.SemaphoreType.DMA,
        ],
    )
    def deg(col_hbm, ew_hbm, out_hbm,
            col_all, ew_all, rowsA, zero_v, acc, semA):
        c = lax.axis_index("c")
        s = lax.axis_index("s")
        wid = c * NS + s

        _zero_acc(zero_v, acc, s, d)
        pltpu.sync_copy(col_hbm.at[wid], col_all)
        pltpu.sync_copy(ew_hbm.at[wid], ew_all)
        plsc.subcore_barrier()

        def fill(rows_v, g):
            ew_row = ew_all.at[g]

            def grp(m, _):
                vec = ew_row[pl.ds(m * L, L)]
                for l in range(L):
                    rows_v.at[m * L + l][pl.ds(0, L)] = _bcast_lane(vec, l)
                return 0

            lax.fori_loop(0, B // L, grp, 0)

        def one(g, _):
            fill(rowsA, g)
            pltpu.sync_copy(rowsA, acc.at[col_all.at[g]], add=True)
            return 0

        lax.fori_loop(0, NCH, one, 0)

        plsc.subcore_barrier()
        _copy_out(acc, out_hbm, c, s)

    return deg


_R = 2000


def _tc_dinv(p0, p1):
    def body(p0_ref, p1_ref, o_ref):
        deg = p0_ref[...][:, 0:1] + p1_ref[...][:, 0:1]
        o_ref[...] = jnp.where(deg > 0, lax.rsqrt(jnp.where(deg > 0, deg, 1.0)), 0.0)

    return pl.pallas_call(
        body,
        grid=(N // _R,),
        in_specs=[pl.BlockSpec((_R, H), lambda i: (i, 0)),
                  pl.BlockSpec((_R, H), lambda i: (i, 0))],
        out_specs=pl.BlockSpec((_R, 1), lambda i: (i, 0)),
        out_shape=jax.ShapeDtypeStruct((N, 1), jnp.float32),
    )(p0, p1)


def _tc_first(x, W, dinv):
    def body(x_ref, w_ref, dv_ref, o_ref):
        o_ref[...] = dv_ref[...] * jnp.dot(
            x_ref[...], w_ref[...], preferred_element_type=jnp.float32)

    return pl.pallas_call(
        body,
        grid=(N // _R,),
        in_specs=[pl.BlockSpec((_R, D_IN), lambda i: (i, 0)),
                  pl.BlockSpec((D_IN, H), lambda i: (0, 0)),
                  pl.BlockSpec((_R, 1), lambda i: (i, 0))],
        out_specs=pl.BlockSpec((_R, H), lambda i: (i, 0)),
        out_shape=jax.ShapeDtypeStruct((N, H), jnp.float32),
    )(x, W, dinv)


def _tc_mid(p0, p1, dinv, b, W):
    dn = W.shape[1]

    def body(p0_ref, p1_ref, dv_ref, b_ref, w_ref, o_ref):
        dv = dv_ref[...]
        h = dv * (p0_ref[...] + p1_ref[...]) + b_ref[...]
        h = jnp.where(h > 0, h, jnp.exp(h) - 1.0)
        o_ref[...] = dv * jnp.dot(h, w_ref[...], preferred_element_type=jnp.float32)

    return pl.pallas_call(
        body,
        grid=(N // _R,),
        in_specs=[pl.BlockSpec((_R, H), lambda i: (i, 0)),
                  pl.BlockSpec((_R, H), lambda i: (i, 0)),
                  pl.BlockSpec((_R, 1), lambda i: (i, 0)),
                  pl.BlockSpec((1, H), lambda i: (0, 0)),
                  pl.BlockSpec((H, dn), lambda i: (0, 0))],
        out_specs=pl.BlockSpec((_R, dn), lambda i: (i, 0)),
        out_shape=jax.ShapeDtypeStruct((N, dn), jnp.float32),
    )(p0, p1, dinv, b, W)


def _tc_last(p0, p1, dinv, b):
    dp = p0.shape[1]

    def body(p0_ref, p1_ref, dv_ref, b_ref, o_ref):
        t = dv_ref[...] * (p0_ref[...] + p1_ref[...]) + b_ref[...]
        o_ref[...] = t[:, :C]

    return pl.pallas_call(
        body,
        grid=(N // _R,),
        in_specs=[pl.BlockSpec((_R, dp), lambda i: (i, 0)),
                  pl.BlockSpec((_R, dp), lambda i: (i, 0)),
                  pl.BlockSpec((_R, 1), lambda i: (i, 0)),
                  pl.BlockSpec((1, dp), lambda i: (0, 0))],
        out_specs=pl.BlockSpec((_R, C), lambda i: (i, 0)),
        out_shape=jax.ShapeDtypeStruct((N, C), jnp.float32),
    )(p0, p1, dinv, b)


_sc_agg128 = _make_sc_agg(128)
_sc_deg = _make_sc_deg()


def kernel(x, edge_index, weight, W1, W2, W3, W4, W5, W6, W7, W8, W9,
           b1, b2, b3, b4, b5, b6, b7, b8, b9):
    loop = jnp.arange(N, dtype=jnp.int32)
    pad = E_PAD - E_TOT
    row0 = jnp.concatenate([edge_index[0].astype(jnp.int32), loop])
    col0 = jnp.concatenate([edge_index[1].astype(jnp.int32), loop])
    ew0 = jnp.concatenate([weight.astype(jnp.float32),
                           jnp.ones((N,), jnp.float32)])
    perm = jnp.argsort(row0)
    row = jnp.concatenate([row0[perm],
                           jnp.zeros((pad,), jnp.int32)]).reshape(NW, NCH, B)
    col = jnp.concatenate([col0[perm],
                           jnp.zeros((pad,), jnp.int32)]).reshape(NW, NCH, B)
    ew = jnp.concatenate([ew0[perm],
                          jnp.zeros((pad,), jnp.float32)]).reshape(NW, NCH, B)

    degp = _sc_deg(col, ew)
    dinv = _tc_dinv(degp[:N], degp[N:])

    Ws = [W2, W3, W4, W5, W6, W7, W8]
    bs = [b1, b2, b3, b4, b5, b6, b7]

    h2 = _tc_first(x, W1, dinv)
    for i in range(7):
        pp = _sc_agg128(row, col, ew, h2)
        h2 = _tc_mid(pp[:N], pp[N:], dinv, bs[i].reshape(1, H), Ws[i])
    pp = _sc_agg128(row, col, ew, h2)
    W9p = jnp.pad(W9, ((0, 0), (0, H - C)))
    h2 = _tc_mid(pp[:N], pp[N:], dinv, b8.reshape(1, H), W9p)

    pp = _sc_agg128(row, col, ew, h2)
    b9p = jnp.pad(b9, (0, H - C)).reshape(1, H)
    return _tc_last(pp[:N], pp[N:], dinv, b9p)


# speedup vs baseline: 1.2068x; 1.2068x over previous
import functools

import jax
import jax.numpy as jnp
from jax import lax
from jax.experimental import pallas as pl
from jax.experimental.pallas import tpu as pltpu
from jax.experimental.pallas import tpu_sc as plsc

N = 10000
D_IN = 128
H = 128
C = 40

NC = 2
NS = 16
NW = NC * NS
L = 16
B = 128
P = 2

RPT = 624
ZR = 24

E_RAW = 320000
E_TOT = E_RAW + N
NCH = P * (-(-E_TOT // (NW * B * P)))
EPW = NCH * B
E_PAD = NW * EPW


_GDN = lax.GatherDimensionNumbers(
    offset_dims=(), collapsed_slice_dims=(0,), start_index_map=(0,))


def _bcast_lane(vec, lane):
    idx = jnp.full((L,), lane, jnp.int32)
    return lax.gather(vec, idx[:, None], _GDN, (1,),
                      mode=lax.GatherScatterMode.PROMISE_IN_BOUNDS)


def _zero_acc(zero_v, acc, s, d):
    z16 = jnp.zeros((L,), jnp.float32)

    def zrow(r, _):
        for k in range(d // L):
            zero_v[r, pl.ds(k * L, L)] = z16
        return 0

    lax.fori_loop(0, ZR, zrow, 0)
    base = s * RPT
    for i in range(RPT // ZR):
        pltpu.sync_copy(zero_v, acc.at[pl.ds(base + i * ZR, ZR)])

    @pl.when(s == NS - 1)
    def _():
        pltpu.sync_copy(zero_v.at[pl.ds(0, N - NS * RPT)],
                        acc.at[pl.ds(NS * RPT, N - NS * RPT)])


def _copy_out(acc, out_hbm, c, s):
    base = s * RPT
    pltpu.sync_copy(acc.at[pl.ds(base, RPT)],
                    out_hbm.at[pl.ds(c * N + base, RPT)])

    @pl.when(s == NS - 1)
    def _():
        tail = N - NS * RPT
        pltpu.sync_copy(acc.at[pl.ds(NS * RPT, tail)],
                        out_hbm.at[pl.ds(c * N + NS * RPT, tail)])


def _make_sc_agg(d):
    mesh = plsc.VectorSubcoreMesh(core_axis_name="c", subcore_axis_name="s")

    def _scale(rows_v, ew_row):
        def grp(m, _):
            vec = ew_row[pl.ds(m * L, L)]
            for l in range(L):
                sc = _bcast_lane(vec, l)
                row = rows_v.at[m * L + l]
                for k in range(d // L):
                    sl = pl.ds(k * L, L)
                    row[sl] = row[sl] * sc
            return 0

        lax.fori_loop(0, B // L, grp, 0)

    @functools.partial(
        pl.kernel,
        out_type=jax.ShapeDtypeStruct((NC * N, d), jnp.float32),
        mesh=mesh,
        scratch_types=[
            pltpu.VMEM((P, B), jnp.int32),
            pltpu.VMEM((P, B), jnp.int32),
            pltpu.VMEM((P, B), jnp.float32),
            pltpu.VMEM((P, B), jnp.int32),
            pltpu.VMEM((P, B), jnp.int32),
            pltpu.VMEM((P, B), jnp.float32),
            pltpu.VMEM((B, d), jnp.float32),
            pltpu.VMEM((B, d), jnp.float32),
            pltpu.VMEM((ZR, d), jnp.float32),
            pltpu.VMEM_SHARED((N, d), jnp.float32),
            pltpu.SemaphoreType.DMA,
            pltpu.SemaphoreType.DMA,
            pltpu.SemaphoreType.DMA,
        ],
    )
    def agg(row_hbm, col_hbm, ew_hbm, x2_hbm, out_hbm,
            rowA, colA, ewA, rowB, colB, ewB, rows0, rows1,
            zero_v, acc, semi, sem0, sem1):
        c = lax.axis_index("c")
        s = lax.axis_index("s")
        wid = c * NS + s

        _zero_acc(zero_v, acc, s, d)
        plsc.subcore_barrier()

        rows = (rows0, rows1)
        sems = (sem0, sem1)
        idxs = ((rowA, colA, ewA), (rowB, colB, ewB))

        def idx_start(blk, slot):
            for src_hbm, dst in zip((row_hbm, col_hbm, ew_hbm), slot):
                pltpu.async_copy(src_hbm.at[wid].at[pl.ds(blk * P, P)], dst,
                                 semi)

        def idx_wait(blk, slot):
            for src_hbm, dst in zip((row_hbm, col_hbm, ew_hbm), slot):
                pltpu.make_async_copy(src_hbm.at[wid].at[pl.ds(blk * P, P)],
                                      dst, semi).wait()

        idx_start(0, idxs[0])
        idx_wait(0, idxs[0])
        for p in range(P):
            pltpu.async_copy(x2_hbm.at[idxs[0][0].at[p]], rows[p], sems[p])
        idx_start(1, idxs[1])

        nblk = NCH // P

        def block(i, _):
            for q in range(2):
                @pl.when(i % 2 == q)
                def _():
                    cur = idxs[q]
                    nxt = idxs[1 - q]
                    for p in range(P):
                        pltpu.make_async_copy(
                            x2_hbm.at[cur[0].at[p]], rows[p], sems[p]).wait()
                        _scale(rows[p], cur[2].at[p])
                        pltpu.sync_copy(rows[p], acc.at[cur[1].at[p]],
                                        add=True)

                        @pl.when(i + 1 < nblk)
                        def _():
                            if p == 0:
                                idx_wait(i + 1, nxt)
                            pltpu.async_copy(
                                x2_hbm.at[nxt[0].at[p]], rows[p], sems[p])

                    @pl.when(i + 2 < nblk)
                    def _():
                        idx_start(i + 2, cur)
            return 0

        lax.fori_loop(0, nblk, block, 0)

        plsc.subcore_barrier()
        _copy_out(acc, out_hbm, c, s)

    return agg


def _make_sc_deg():
    d = H
    mesh = plsc.VectorSubcoreMesh(core_axis_name="c", subcore_axis_name="s")

    @functools.partial(
        pl.kernel,
        out_type=jax.ShapeDtypeStruct((NC * N, d), jnp.float32),
        mesh=mesh,
        scratch_types=[
            pltpu.VMEM((NCH, B), jnp.int32),
            pltpu.VMEM((NCH, B), jnp.float32),
            pltpu.VMEM((B, d), jnp.float32),
            pltpu.VMEM((ZR, d), jnp.float32),
            pltpu.VMEM_SHARED((N, d), jnp.float32),
            pltpu.SemaphoreType.DMA,
        ],
    )
    def deg(col_hbm, ew_hbm, out_hbm,
            col_all, ew_all, rowsA, zero_v, acc, semA):
        c = lax.axis_index("c")
        s = lax.axis_index("s")
        wid = c * NS + s

        _zero_acc(zero_v, acc, s, d)
        pltpu.sync_copy(col_hbm.at[wid], col_all)
        pltpu.sync_copy(ew_hbm.at[wid], ew_all)
        plsc.subcore_barrier()

        def fill(rows_v, g):
            ew_row = ew_all.at[g]

            def grp(m, _):
                vec = ew_row[pl.ds(m * L, L)]
                for l in range(L):
                    rows_v.at[m * L + l][pl.ds(0, L)] = _bcast_lane(vec, l)
                return 0

            lax.fori_loop(0, B // L, grp, 0)

        def one(g, _):
            fill(rowsA, g)
            pltpu.sync_copy(rowsA, acc.at[col_all.at[g]], add=True)
            return 0

        lax.fori_loop(0, NCH, one, 0)

        plsc.subcore_barrier()
        _copy_out(acc, out_hbm, c, s)

    return deg


_R = 2000


def _tc_dinv(p0, p1):
    def body(p0_ref, p1_ref, o_ref):
        deg = p0_ref[...][:, 0:1] + p1_ref[...][:, 0:1]
        o_ref[...] = jnp.where(deg > 0, lax.rsqrt(jnp.where(deg > 0, deg, 1.0)), 0.0)

    return pl.pallas_call(
        body,
        grid=(N // _R,),
        in_specs=[pl.BlockSpec((_R, H), lambda i: (i, 0)),
                  pl.BlockSpec((_R, H), lambda i: (i, 0))],
        out_specs=pl.BlockSpec((_R, 1), lambda i: (i, 0)),
        out_shape=jax.ShapeDtypeStruct((N, 1), jnp.float32),
    )(p0, p1)


def _tc_first(x, W, dinv):
    def body(x_ref, w_ref, dv_ref, o_ref):
        o_ref[...] = dv_ref[...] * jnp.dot(
            x_ref[...], w_ref[...], preferred_element_type=jnp.float32)

    return pl.pallas_call(
        body,
        grid=(N // _R,),
        in_specs=[pl.BlockSpec((_R, D_IN), lambda i: (i, 0)),
                  pl.BlockSpec((D_IN, H), lambda i: (0, 0)),
                  pl.BlockSpec((_R, 1), lambda i: (i, 0))],
        out_specs=pl.BlockSpec((_R, H), lambda i: (i, 0)),
        out_shape=jax.ShapeDtypeStruct((N, H), jnp.float32),
    )(x, W, dinv)


def _tc_mid(p0, p1, dinv, b, W):
    dn = W.shape[1]

    def body(p0_ref, p1_ref, dv_ref, b_ref, w_ref, o_ref):
        dv = dv_ref[...]
        h = dv * (p0_ref[...] + p1_ref[...]) + b_ref[...]
        h = jnp.where(h > 0, h, jnp.exp(h) - 1.0)
        o_ref[...] = dv * jnp.dot(h, w_ref[...], preferred_element_type=jnp.float32)

    return pl.pallas_call(
        body,
        grid=(N // _R,),
        in_specs=[pl.BlockSpec((_R, H), lambda i: (i, 0)),
                  pl.BlockSpec((_R, H), lambda i: (i, 0)),
                  pl.BlockSpec((_R, 1), lambda i: (i, 0)),
                  pl.BlockSpec((1, H), lambda i: (0, 0)),
                  pl.BlockSpec((H, dn), lambda i: (0, 0))],
        out_specs=pl.BlockSpec((_R, dn), lambda i: (i, 0)),
        out_shape=jax.ShapeDtypeStruct((N, dn), jnp.float32),
    )(p0, p1, dinv, b, W)


def _tc_last(p0, p1, dinv, b):
    dp = p0.shape[1]

    def body(p0_ref, p1_ref, dv_ref, b_ref, o_ref):
        t = dv_ref[...] * (p0_ref[...] + p1_ref[...]) + b_ref[...]
        o_ref[...] = t[:, :C]

    return pl.pallas_call(
        body,
        grid=(N // _R,),
        in_specs=[pl.BlockSpec((_R, dp), lambda i: (i, 0)),
                  pl.BlockSpec((_R, dp), lambda i: (i, 0)),
                  pl.BlockSpec((_R, 1), lambda i: (i, 0)),
                  pl.BlockSpec((1, dp), lambda i: (0, 0))],
        out_specs=pl.BlockSpec((_R, C), lambda i: (i, 0)),
        out_shape=jax.ShapeDtypeStruct((N, C), jnp.float32),
    )(p0, p1, dinv, b)


_sc_agg128 = _make_sc_agg(128)
_sc_deg = _make_sc_deg()


def kernel(x, edge_index, weight, W1, W2, W3, W4, W5, W6, W7, W8, W9,
           b1, b2, b3, b4, b5, b6, b7, b8, b9):
    loop = jnp.arange(N, dtype=jnp.int32)
    pad = E_PAD - E_TOT
    row0 = jnp.concatenate([edge_index[0].astype(jnp.int32), loop])
    col0 = jnp.concatenate([edge_index[1].astype(jnp.int32), loop])
    ew0 = jnp.concatenate([weight.astype(jnp.float32),
                           jnp.ones((N,), jnp.float32)])
    row = jnp.concatenate([row0,
                           jnp.zeros((pad,), jnp.int32)]).reshape(NW, NCH, B)
    col = jnp.concatenate([col0,
                           jnp.zeros((pad,), jnp.int32)]).reshape(NW, NCH, B)
    ew = jnp.concatenate([ew0,
                          jnp.zeros((pad,), jnp.float32)]).reshape(NW, NCH, B)

    degp = _sc_deg(col, ew)
    dinv = _tc_dinv(degp[:N], degp[N:])

    Ws = [W2, W3, W4, W5, W6, W7, W8]
    bs = [b1, b2, b3, b4, b5, b6, b7]

    h2 = _tc_first(x, W1, dinv)
    for i in range(7):
        pp = _sc_agg128(row, col, ew, h2)
        h2 = _tc_mid(pp[:N], pp[N:], dinv, bs[i].reshape(1, H), Ws[i])
    pp = _sc_agg128(row, col, ew, h2)
    W9p = jnp.pad(W9, ((0, 0), (0, H - C)))
    h2 = _tc_mid(pp[:N], pp[N:], dinv, b8.reshape(1, H), W9p)

    pp = _sc_agg128(row, col, ew, h2)
    b9p = jnp.pad(b9, (0, H - C)).reshape(1, H)
    return _tc_last(pp[:N], pp[N:], dinv, b9p)
